# speedup vs baseline: 2.0814x; 2.0814x over previous
import jax
import jax.numpy as jnp
from jax.experimental import pallas as pl
from jax.experimental.pallas import tpu as pltpu

S_BLK = 256


def _add_body(x_ref, pos_ref, o_ref):
    o_ref[...] = x_ref[...] + pos_ref[...]


def kernel(x, pos_embedding):
    batch, seq_len, d_model = x.shape
    xf = x.reshape(batch * seq_len, d_model)
    n_blocks = (batch * seq_len) // S_BLK
    pos_blocks = seq_len // S_BLK

    out = pl.pallas_call(
        _add_body,
        grid=(n_blocks,),
        in_specs=[
            pl.BlockSpec((S_BLK, d_model), lambda i: (i, 0)),
            pl.BlockSpec((S_BLK, d_model), lambda i: (i % pos_blocks, 0)),
        ],
        out_specs=pl.BlockSpec((S_BLK, d_model), lambda i: (i, 0)),
        out_shape=jax.ShapeDtypeStruct((batch * seq_len, d_model), x.dtype),
    )(xf, pos_embedding)
    return out.reshape(batch, seq_len, d_model)


# speedup vs baseline: 2.1890x; 1.0517x over previous
import jax
import jax.numpy as jnp
from jax.experimental import pallas as pl
from jax.experimental.pallas import tpu as pltpu

S_BLK = 256


def _add_body(x_ref, pos_ref, o_ref):
    o_ref[...] = x_ref[...] + pos_ref[...]


def kernel(x, pos_embedding):
    batch, seq_len, d_model = x.shape
    xf = x.reshape(batch * seq_len, d_model)
    n_blocks = (batch * seq_len) // S_BLK
    pos_blocks = seq_len // S_BLK

    out = pl.pallas_call(
        _add_body,
        grid=(pos_blocks, batch),
        in_specs=[
            pl.BlockSpec((S_BLK, d_model), lambda i, b: (b * pos_blocks + i, 0)),
            pl.BlockSpec((S_BLK, d_model), lambda i, b: (i, 0)),
        ],
        out_specs=pl.BlockSpec((S_BLK, d_model), lambda i, b: (b * pos_blocks + i, 0)),
        out_shape=jax.ShapeDtypeStruct((batch * seq_len, d_model), x.dtype),
    )(xf, pos_embedding)
    return out.reshape(batch, seq_len, d_model)
